# baseline (speedup 1.0000x reference)
import jax
import jax.numpy as jnp
from jax import lax
from jax.experimental import pallas as pl
from jax.experimental.pallas import tpu as pltpu
from jax.experimental.pallas import tpu_sc as plsc

B = 16384
D = 32
NC = 2
NS = 16
NW = NC * NS
BPW = B // NW
CH = 128
NCH = BPW // CH


def _body(users_hbm, movies_hbm, utab_hbm, mtab_hbm, wb_hbm, out_hbm,
          uflat_v, mflat_v, ugidx_v, mgidx_v, ubuf_v, mbuf_v, wb_v, out_v,
          sem):
    wid = lax.axis_index("s") * NC + lax.axis_index("c")
    base = wid * BPW

    pltpu.sync_copy(users_hbm.at[pl.ds(base, BPW)], uflat_v)
    pltpu.sync_copy(movies_hbm.at[pl.ds(base, BPW)], mflat_v)
    pltpu.sync_copy(wb_hbm, wb_v)

    def mkidx(k, carry):
        ugidx_v[pl.ds(k * 16, 16)] = lax.shift_right_logical(
            uflat_v[pl.ds(k * 16, 16)], 2)
        mgidx_v[pl.ds(k * 16, 16)] = lax.shift_right_logical(
            mflat_v[pl.ds(k * 16, 16)], 2)
        return carry

    lax.fori_loop(0, BPW // 16, mkidx, None)

    lanes = lax.iota(jnp.int32, 16)
    bias = wb_v[pl.ds(2 * D * 16, 16)]

    def chunk(j, carry):
        cu_copy = pltpu.async_copy(
            utab_hbm.at[ugidx_v.at[pl.ds(j * CH, CH)]], ubuf_v, sem)
        cm_copy = pltpu.async_copy(
            mtab_hbm.at[mgidx_v.at[pl.ds(j * CH, CH)]], mbuf_v, sem)
        cu_copy.wait()
        cm_copy.wait()

        def group(g, carry2):
            off = j * CH + g * 16
            rows = lanes + g * 16
            coffu = lax.shift_left(uflat_v[pl.ds(off, 16)] & 3, 5)
            coffm = lax.shift_left(mflat_v[pl.ds(off, 16)] & 3, 5)
            acc = bias
            for d in range(D):
                cu = plsc.load_gather(ubuf_v, [rows, coffu + d])
                cm = plsc.load_gather(mbuf_v, [rows, coffm + d])
                acc = (acc + cu * wb_v[pl.ds(d * 16, 16)]
                       + cm * wb_v[pl.ds((D + d) * 16, 16)])
            out_v[pl.ds(off, 16)] = acc
            return carry2

        lax.fori_loop(0, CH // 16, group, None)
        return carry

    lax.fori_loop(0, NCH, chunk, None)
    pltpu.sync_copy(out_v, out_hbm.at[pl.ds(base, BPW)])


def kernel(users, movies, user_table, movie_table, W, b):
    utab = user_table.reshape(-1, 4 * D)
    mtab = movie_table.reshape(-1, 4 * D)
    wb = (jnp.concatenate([W[:, 0], b])[:, None]
          * jnp.ones((1, 16), jnp.float32)).reshape(-1)
    mesh = plsc.VectorSubcoreMesh(core_axis_name="c", subcore_axis_name="s")
    out = pl.kernel(
        _body,
        mesh=mesh,
        out_type=jax.ShapeDtypeStruct((B,), jnp.float32),
        compiler_params=pltpu.CompilerParams(
            needs_layout_passes=False, use_tc_tiling_on_sc=True),
        scratch_types=[
            pltpu.VMEM((BPW,), jnp.int32),
            pltpu.VMEM((BPW,), jnp.int32),
            pltpu.VMEM((BPW,), jnp.int32),
            pltpu.VMEM((BPW,), jnp.int32),
            pltpu.VMEM((CH, 4 * D), jnp.float32),
            pltpu.VMEM((CH, 4 * D), jnp.float32),
            pltpu.VMEM(((2 * D + 1) * 16,), jnp.float32),
            pltpu.VMEM((BPW,), jnp.float32),
            pltpu.SemaphoreType.DMA,
        ],
    )(users, movies, utab, mtab, wb)
    return out.reshape(B, 1)


# speedup vs baseline: 7.4481x; 7.4481x over previous
import functools

import jax
import jax.numpy as jnp
from jax import lax
from jax.experimental import pallas as pl
from jax.experimental.pallas import tpu as pltpu
from jax.experimental.pallas import tpu_sc as plsc

B = 16384
D = 32
NC = 2
NS = 16
NW = NC * NS
BPW = B // NW
CH = 128
NCH = BPW // CH


def _proj_body(w_ref, xT_ref, o_ref):
    o_ref[...] = jnp.sum(xT_ref[...] * w_ref[...], axis=0)


def _project(xT, w, bc):
    n = xT.shape[1]
    grid = (n + bc - 1) // bc
    return pl.pallas_call(
        _proj_body,
        grid=(grid,),
        in_specs=[
            pl.BlockSpec((D, 1), lambda i: (0, 0)),
            pl.BlockSpec((D, bc), lambda i: (0, i)),
        ],
        out_specs=pl.BlockSpec((bc,), lambda i: (i,)),
        out_shape=jax.ShapeDtypeStruct((n,), jnp.float32),
    )(w, xT)


def _gather_body(users_hbm, movies_hbm, uproj_hbm, mproj_hbm, bias_hbm,
                 out_hbm, uix_v, mix_v, uval_v, mval_v, bias_v, out_v, sem):
    wid = lax.axis_index("s") * NC + lax.axis_index("c")
    base = wid * BPW

    pltpu.sync_copy(users_hbm.at[pl.ds(base, BPW)], uix_v)
    pltpu.sync_copy(movies_hbm.at[pl.ds(base, BPW)], mix_v)
    pltpu.sync_copy(bias_hbm, bias_v)

    copies = []
    for j in range(NCH):
        sl = pl.ds(j * CH, CH)
        copies.append(pltpu.async_copy(
            uproj_hbm.at[uix_v.at[sl]], uval_v.at[sl], sem))
        copies.append(pltpu.async_copy(
            mproj_hbm.at[mix_v.at[sl]], mval_v.at[sl], sem))
    for c in copies:
        c.wait()

    bias = bias_v[...]

    def group(g, carry):
        sl = pl.ds(g * 16, 16)
        out_v[sl] = uval_v[sl] + mval_v[sl] + bias
        return carry

    lax.fori_loop(0, BPW // 16, group, None)
    pltpu.sync_copy(out_v, out_hbm.at[pl.ds(base, BPW)])


def kernel(users, movies, user_table, movie_table, W, b):
    utabT = user_table.T
    mtabT = movie_table.T
    u_proj = _project(utabT, W[:D], 65536)
    m_proj = _project(mtabT, W[D:], 32768)
    b16 = jnp.broadcast_to(b, (16,))

    mesh = plsc.VectorSubcoreMesh(core_axis_name="c", subcore_axis_name="s")
    out = pl.kernel(
        _gather_body,
        mesh=mesh,
        out_type=jax.ShapeDtypeStruct((B,), jnp.float32),
        compiler_params=pltpu.CompilerParams(needs_layout_passes=False),
        scratch_types=[
            pltpu.VMEM((BPW,), jnp.int32),
            pltpu.VMEM((BPW,), jnp.int32),
            pltpu.VMEM((BPW,), jnp.float32),
            pltpu.VMEM((BPW,), jnp.float32),
            pltpu.VMEM((16,), jnp.float32),
            pltpu.VMEM((BPW,), jnp.float32),
            pltpu.SemaphoreType.DMA,
        ],
    )(users, movies, u_proj, m_proj, b16)
    return out.reshape(B, 1)
